# baseline (speedup 1.0000x reference)
import jax
import jax.numpy as jnp
from jax import lax
from jax.experimental import pallas as pl
from jax.experimental.pallas import tpu as pltpu
from jax.experimental.pallas import tpu_sc as plsc

N = 10000
E = 320000
D = 128
NC = 2
NS = 16
NW = NC * NS
L = 16
CHUNK = 128
EPT = -(-E // (NW * 3 * CHUNK)) * 3 * CHUNK
E_PAD = EPT * NW
NCH = EPT // CHUNK
N_PAD = 10112
NACC = 10112
INIT_ROWS = NACC // NS
OUT_ROWS = N_PAD // NS
BLK = 1264


def _mesh():
  return plsc.VectorSubcoreMesh(
      core_axis_name="c", subcore_axis_name="s", num_cores=NC, num_subcores=NS)


def _deg(dst3, z128, ones128):
  def body(dst3, z128, ones128, out, acc, dst_v, ones_v, sem):
    c = lax.axis_index("c")
    s = lax.axis_index("s")
    wid = s * NC + c
    r0 = s * INIT_ROWS
    pltpu.sync_copy(z128.at[pl.ds(r0, INIT_ROWS)], acc.at[pl.ds(r0, INIT_ROWS)])
    pltpu.sync_copy(ones128, ones_v)
    pltpu.sync_copy(dst3.at[wid], dst_v)
    plsc.subcore_barrier()

    def step(g, carry):
      pltpu.sync_copy(ones_v, acc.at[dst_v.at[g]], add=True)
      return carry

    lax.fori_loop(0, NCH, step, 0)
    plsc.subcore_barrier()
    o0 = s * OUT_ROWS
    pltpu.sync_copy(acc.at[pl.ds(o0, OUT_ROWS)], out.at[c, pl.ds(o0, OUT_ROWS)])

  return pl.kernel(
      body,
      out_type=jax.ShapeDtypeStruct((NC, N_PAD, D), jnp.float32),
      mesh=_mesh(),
      scratch_types=[
          pltpu.VMEM_SHARED((NACC, D), jnp.float32),
          pltpu.VMEM((NCH, CHUNK), jnp.int32),
          pltpu.VMEM((CHUNK, D), jnp.float32),
          pltpu.SemaphoreType.DMA,
      ],
  )(dst3, z128, ones128)


def _agg(table, src3, dst3, z128):
  def body(table, src3, dst3, z128, out, acc, src2, dst2,
           rows0, rows1, rows2, sem0, sem1, sem2):
    c = lax.axis_index("c")
    s = lax.axis_index("s")
    wid = s * NC + c
    r0 = s * INIT_ROWS
    pltpu.sync_copy(z128.at[pl.ds(r0, INIT_ROWS)], acc.at[pl.ds(r0, INIT_ROWS)])
    plsc.subcore_barrier()
    rows = (rows0, rows1, rows2)
    sem = (sem0, sem1, sem2)

    for j in range(2):
      pltpu.sync_copy(src3.at[wid, j], src2.at[j])
      pltpu.sync_copy(dst3.at[wid, j], dst2.at[j])
      pltpu.make_async_copy(table.at[src2.at[j]], rows[j], sem[j]).start()

    def outer(g3, carry):
      for b in range(3):
        g = g3 * 3 + b
        nb = (b + 2) % 3

        pltpu.make_async_copy(table.at[src2.at[b]], rows[b], sem[b]).wait()

        @pl.when(g + 2 < NCH)
        def _prefire():
          pltpu.sync_copy(src3.at[wid, g + 2], src2.at[nb])
          pltpu.sync_copy(dst3.at[wid, g + 2], dst2.at[nb])
          pltpu.make_async_copy(table.at[src2.at[nb]], rows[nb], sem[nb]).start()

        pltpu.sync_copy(rows[b], acc.at[dst2.at[b]], add=True)
      return carry

    lax.fori_loop(0, NCH // 3, outer, 0)
    plsc.subcore_barrier()
    o0 = s * OUT_ROWS
    pltpu.sync_copy(acc.at[pl.ds(o0, OUT_ROWS)], out.at[c, pl.ds(o0, OUT_ROWS)])

  return pl.kernel(
      body,
      out_type=jax.ShapeDtypeStruct((NC, N_PAD, D), jnp.float32),
      mesh=_mesh(),
      scratch_types=[
          pltpu.VMEM_SHARED((NACC, D), jnp.float32),
          pltpu.VMEM((3, CHUNK), jnp.int32),
          pltpu.VMEM((3, CHUNK), jnp.int32),
          pltpu.VMEM((CHUNK, D), jnp.float32),
          pltpu.VMEM((CHUNK, D), jnp.float32),
          pltpu.VMEM((CHUNK, D), jnp.float32),
          pltpu.SemaphoreType.DMA,
          pltpu.SemaphoreType.DMA,
          pltpu.SemaphoreType.DMA,
      ],
  )(table, src3, dst3, z128)


def _tc_layer1(x, na, nb, da, db, Ws, Wn, bs, bn):
  def body(x_r, na_r, nb_r, da_r, db_r, ws_r, wn_r, b_r, out_r):
    deg = da_r[:, 0:1] + db_r[:, 0:1]
    inv = 1.0 / jnp.maximum(deg, 1.0)
    neigh = (na_r[...] + nb_r[...]) * inv
    h = (jnp.dot(x_r[...], ws_r[...], preferred_element_type=jnp.float32)
         + jnp.dot(neigh, wn_r[...], preferred_element_type=jnp.float32)
         + b_r[...])
    out_r[...] = jnp.maximum(h, 0.0)

  row = lambda i: (i, 0)
  fix = lambda i: (0, 0)
  return pl.pallas_call(
      body,
      grid=(N_PAD // BLK,),
      in_specs=[
          pl.BlockSpec((BLK, D), row),
          pl.BlockSpec((BLK, D), row),
          pl.BlockSpec((BLK, D), row),
          pl.BlockSpec((BLK, D), row),
          pl.BlockSpec((BLK, D), row),
          pl.BlockSpec((D, D), fix),
          pl.BlockSpec((D, D), fix),
          pl.BlockSpec((1, D), fix),
      ],
      out_specs=pl.BlockSpec((BLK, D), row),
      out_shape=jax.ShapeDtypeStruct((N_PAD, D), jnp.float32),
  )(x, na, nb, da, db, Ws, Wn, (bs + bn).reshape(1, D))


def _tc_layer2(h1, na, nb, da, db, Ws, Wn, bs, bn, Wpc, bvec):
  def body(h_r, na_r, nb_r, da_r, db_r, ws_r, wn_r, b_r, wp_r, bv_r, out_r):
    deg = da_r[:, 0:1] + db_r[:, 0:1]
    inv = 1.0 / jnp.maximum(deg, 1.0)
    neigh = (na_r[...] + nb_r[...]) * inv
    h2 = (jnp.dot(h_r[...], ws_r[...], preferred_element_type=jnp.float32)
          + jnp.dot(neigh, wn_r[...], preferred_element_type=jnp.float32)
          + b_r[...])
    out_r[...] = jnp.dot(h2, wp_r[...], preferred_element_type=jnp.float32) + bv_r[...]

  row = lambda i: (i, 0)
  fix = lambda i: (0, 0)
  return pl.pallas_call(
      body,
      grid=(N_PAD // BLK,),
      in_specs=[
          pl.BlockSpec((BLK, D), row),
          pl.BlockSpec((BLK, D), row),
          pl.BlockSpec((BLK, D), row),
          pl.BlockSpec((BLK, D), row),
          pl.BlockSpec((BLK, D), row),
          pl.BlockSpec((D, D), fix),
          pl.BlockSpec((D, D), fix),
          pl.BlockSpec((1, D), fix),
          pl.BlockSpec((D, 2), fix),
          pl.BlockSpec((1, 2), fix),
      ],
      out_specs=pl.BlockSpec((BLK, 2), row),
      out_shape=jax.ShapeDtypeStruct((N_PAD, 2), jnp.float32),
  )(h1, na, nb, da, db, Ws, Wn, (bs + bn).reshape(1, D), Wpc, bvec)


def _edge_score(ab, srcb, dstb):
  def body(ab, srcb, dstb, out, ab_v, src_v, dst_v, out_v):
    c = lax.axis_index("c")
    s = lax.axis_index("s")
    wid = s * NC + c
    base = wid * EPT
    pltpu.sync_copy(ab, ab_v)
    pltpu.sync_copy(srcb.at[pl.ds(base, EPT)], src_v)
    pltpu.sync_copy(dstb.at[pl.ds(base, EPT)], dst_v)

    def step(i, carry):
      si = src_v[pl.ds(i * L, L)]
      di = dst_v[pl.ds(i * L, L)]
      av = plsc.load_gather(ab_v, [si * 2])
      bv = plsc.load_gather(ab_v, [di * 2 + 1])
      out_v[pl.ds(i * L, L)] = av + bv
      return carry

    lax.fori_loop(0, EPT // L, step, 0)
    pltpu.sync_copy(out_v, out.at[pl.ds(base, EPT)])

  return pl.kernel(
      body,
      out_type=jax.ShapeDtypeStruct((E_PAD,), jnp.float32),
      mesh=_mesh(),
      compiler_params=pltpu.CompilerParams(needs_layout_passes=False),
      scratch_types=[
          pltpu.VMEM((N_PAD * 2,), jnp.float32),
          pltpu.VMEM((EPT,), jnp.int32),
          pltpu.VMEM((EPT,), jnp.int32),
          pltpu.VMEM((EPT,), jnp.float32),
      ],
  )(ab, srcb, dstb)


def kernel(x, edge_index, W1s, b1s, W1n, b1n, W2s, b2s, W2n, b2n, Wp, bp):
  src = edge_index[0]
  dst = edge_index[1]
  pad = E_PAD - E
  pad_i = jnp.arange(pad, dtype=jnp.int32)
  srcb = jnp.concatenate([src, pad_i % N])
  dstb = jnp.concatenate([dst, N + pad_i % (NACC - N)])
  z128 = jnp.zeros((NACC, D), jnp.float32)
  ones128 = jnp.ones((CHUNK, D), jnp.float32)

  x_pad = jnp.concatenate([x, jnp.zeros((N_PAD - N, D), jnp.float32)])
  src3 = srcb.reshape(NW, NCH, CHUNK)
  dst3 = dstb.reshape(NW, NCH, CHUNK)
  degp = _deg(dst3, z128, ones128)
  n1p = _agg(x, src3, dst3, z128)
  h1 = _tc_layer1(x_pad, n1p[0], n1p[1], degp[0], degp[1], W1s, W1n, b1s, b1n)
  n2p = _agg(h1, src3, dst3, z128)
  Wpc = jnp.concatenate([Wp[:D], Wp[D:]], axis=1)
  bvec = jnp.concatenate([jnp.zeros((1,), jnp.float32), bp]).reshape(1, 2)
  ab = _tc_layer2(h1, n2p[0], n2p[1], degp[0], degp[1], W2s, W2n, b2s, b2n,
                  Wpc, bvec)
  score = _edge_score(ab.reshape(N_PAD * 2), srcb, dstb)
  return score[:E].reshape(E, 1)


# speedup vs baseline: 1.0993x; 1.0993x over previous
import jax
import jax.numpy as jnp
from jax import lax
from jax.experimental import pallas as pl
from jax.experimental.pallas import tpu as pltpu
from jax.experimental.pallas import tpu_sc as plsc

N = 10000
E = 320000
D = 128
NC = 2
NS = 16
NW = NC * NS
L = 16
CHUNK = 128
EPT = -(-E // (NW * 2 * CHUNK)) * 2 * CHUNK
E_PAD = EPT * NW
NCH = EPT // CHUNK
N_PAD = 10112
NACC = 10240
INIT_ROWS = NACC // NS
OUT_ROWS = N_PAD // NS
BLK = 1264


def _mesh():
  return plsc.VectorSubcoreMesh(
      core_axis_name="c", subcore_axis_name="s", num_cores=NC, num_subcores=NS)


def _deg(dst3, z128, ones128):
  def body(dst3, z128, ones128, out, acc, dst_v, ones_v, sem):
    c = lax.axis_index("c")
    s = lax.axis_index("s")
    wid = s * NC + c
    r0 = s * INIT_ROWS
    pltpu.sync_copy(z128.at[pl.ds(r0, INIT_ROWS)], acc.at[pl.ds(r0, INIT_ROWS)])
    pltpu.sync_copy(ones128, ones_v)
    pltpu.sync_copy(dst3.at[wid], dst_v)
    plsc.subcore_barrier()

    def step(g, carry):
      pltpu.sync_copy(ones_v, acc.at[dst_v.at[g]], add=True)
      return carry

    lax.fori_loop(0, NCH, step, 0)
    plsc.subcore_barrier()
    o0 = s * OUT_ROWS
    pltpu.sync_copy(acc.at[pl.ds(o0, OUT_ROWS)], out.at[c, pl.ds(o0, OUT_ROWS)])

  return pl.kernel(
      body,
      out_type=jax.ShapeDtypeStruct((NC, N_PAD, D), jnp.float32),
      mesh=_mesh(),
      scratch_types=[
          pltpu.VMEM_SHARED((NACC, D), jnp.float32),
          pltpu.VMEM((NCH, CHUNK), jnp.int32),
          pltpu.VMEM((CHUNK, D), jnp.float32),
          pltpu.SemaphoreType.DMA,
      ],
  )(dst3, z128, ones128)


def _agg(table, src3, dst3, z128):
  def body(table, src3, dst3, z128, out, acc, src2, dst_v, rows0, rows1,
           sem0, sem1):
    c = lax.axis_index("c")
    s = lax.axis_index("s")
    wid = s * NC + c
    r0 = s * INIT_ROWS
    pltpu.sync_copy(z128.at[pl.ds(r0, INIT_ROWS)], acc.at[pl.ds(r0, INIT_ROWS)])
    pltpu.sync_copy(dst3.at[wid], dst_v)
    plsc.subcore_barrier()

    rows = (rows0, rows1)
    sem = (sem0, sem1)
    pltpu.sync_copy(src3.at[wid, 0], src2.at[0])
    pltpu.make_async_copy(table.at[src2.at[0]], rows[0], sem[0]).start()

    def outer(g2, carry):
      for b in range(2):
        g = g2 * 2 + b
        nb = 1 - b

        @pl.when(g + 1 < NCH)
        def _load_next_idx():
          pltpu.sync_copy(src3.at[wid, g + 1], src2.at[nb])

        pltpu.make_async_copy(table.at[src2.at[b]], rows[b], sem[b]).wait()

        @pl.when(g + 1 < NCH)
        def _prefire():
          pltpu.make_async_copy(table.at[src2.at[nb]], rows[nb], sem[nb]).start()

        pltpu.sync_copy(rows[b], acc.at[dst_v.at[g]], add=True)
      return carry

    lax.fori_loop(0, NCH // 2, outer, 0)
    plsc.subcore_barrier()
    o0 = s * OUT_ROWS
    pltpu.sync_copy(acc.at[pl.ds(o0, OUT_ROWS)], out.at[c, pl.ds(o0, OUT_ROWS)])

  return pl.kernel(
      body,
      out_type=jax.ShapeDtypeStruct((NC, N_PAD, D), jnp.float32),
      mesh=_mesh(),
      scratch_types=[
          pltpu.VMEM_SHARED((NACC, D), jnp.float32),
          pltpu.VMEM((2, CHUNK), jnp.int32),
          pltpu.VMEM((NCH, CHUNK), jnp.int32),
          pltpu.VMEM((CHUNK, D), jnp.float32),
          pltpu.VMEM((CHUNK, D), jnp.float32),
          pltpu.SemaphoreType.DMA,
          pltpu.SemaphoreType.DMA,
      ],
  )(table, src3, dst3, z128)


def _tc_layer1(x, na, nb, da, db, Ws, Wn, bs, bn):
  def body(x_r, na_r, nb_r, da_r, db_r, ws_r, wn_r, b_r, out_r):
    deg = da_r[:, 0:1] + db_r[:, 0:1]
    inv = 1.0 / jnp.maximum(deg, 1.0)
    neigh = (na_r[...] + nb_r[...]) * inv
    h = (jnp.dot(x_r[...], ws_r[...], preferred_element_type=jnp.float32)
         + jnp.dot(neigh, wn_r[...], preferred_element_type=jnp.float32)
         + b_r[...])
    out_r[...] = jnp.maximum(h, 0.0)

  row = lambda i: (i, 0)
  fix = lambda i: (0, 0)
  return pl.pallas_call(
      body,
      grid=(N_PAD // BLK,),
      in_specs=[
          pl.BlockSpec((BLK, D), row),
          pl.BlockSpec((BLK, D), row),
          pl.BlockSpec((BLK, D), row),
          pl.BlockSpec((BLK, D), row),
          pl.BlockSpec((BLK, D), row),
          pl.BlockSpec((D, D), fix),
          pl.BlockSpec((D, D), fix),
          pl.BlockSpec((1, D), fix),
      ],
      out_specs=pl.BlockSpec((BLK, D), row),
      out_shape=jax.ShapeDtypeStruct((N_PAD, D), jnp.float32),
  )(x, na, nb, da, db, Ws, Wn, (bs + bn).reshape(1, D))


def _tc_layer2(h1, na, nb, da, db, Ws, Wn, bs, bn, Wpc, bvec):
  def body(h_r, na_r, nb_r, da_r, db_r, ws_r, wn_r, b_r, wp_r, bv_r, out_r):
    deg = da_r[:, 0:1] + db_r[:, 0:1]
    inv = 1.0 / jnp.maximum(deg, 1.0)
    neigh = (na_r[...] + nb_r[...]) * inv
    h2 = (jnp.dot(h_r[...], ws_r[...], preferred_element_type=jnp.float32)
          + jnp.dot(neigh, wn_r[...], preferred_element_type=jnp.float32)
          + b_r[...])
    out_r[...] = jnp.dot(h2, wp_r[...], preferred_element_type=jnp.float32) + bv_r[...]

  row = lambda i: (i, 0)
  fix = lambda i: (0, 0)
  return pl.pallas_call(
      body,
      grid=(N_PAD // BLK,),
      in_specs=[
          pl.BlockSpec((BLK, D), row),
          pl.BlockSpec((BLK, D), row),
          pl.BlockSpec((BLK, D), row),
          pl.BlockSpec((BLK, D), row),
          pl.BlockSpec((BLK, D), row),
          pl.BlockSpec((D, D), fix),
          pl.BlockSpec((D, D), fix),
          pl.BlockSpec((1, D), fix),
          pl.BlockSpec((D, 2), fix),
          pl.BlockSpec((1, 2), fix),
      ],
      out_specs=pl.BlockSpec((BLK, 2), row),
      out_shape=jax.ShapeDtypeStruct((N_PAD, 2), jnp.float32),
  )(h1, na, nb, da, db, Ws, Wn, (bs + bn).reshape(1, D), Wpc, bvec)


def _edge_score(ab, srcb, dstb):
  def body(ab, srcb, dstb, out, ab_v, src_v, dst_v, out_v):
    c = lax.axis_index("c")
    s = lax.axis_index("s")
    wid = s * NC + c
    base = wid * EPT
    pltpu.sync_copy(ab, ab_v)
    pltpu.sync_copy(srcb.at[pl.ds(base, EPT)], src_v)
    pltpu.sync_copy(dstb.at[pl.ds(base, EPT)], dst_v)

    def step(i, carry):
      si = src_v[pl.ds(i * L, L)]
      di = dst_v[pl.ds(i * L, L)]
      av = plsc.load_gather(ab_v, [si * 2])
      bv = plsc.load_gather(ab_v, [di * 2 + 1])
      out_v[pl.ds(i * L, L)] = av + bv
      return carry

    lax.fori_loop(0, EPT // L, step, 0)
    pltpu.sync_copy(out_v, out.at[pl.ds(base, EPT)])

  return pl.kernel(
      body,
      out_type=jax.ShapeDtypeStruct((E_PAD,), jnp.float32),
      mesh=_mesh(),
      compiler_params=pltpu.CompilerParams(needs_layout_passes=False),
      scratch_types=[
          pltpu.VMEM((N_PAD * 2,), jnp.float32),
          pltpu.VMEM((EPT,), jnp.int32),
          pltpu.VMEM((EPT,), jnp.int32),
          pltpu.VMEM((EPT,), jnp.float32),
      ],
  )(ab, srcb, dstb)


def kernel(x, edge_index, W1s, b1s, W1n, b1n, W2s, b2s, W2n, b2n, Wp, bp):
  src = edge_index[0]
  dst = edge_index[1]
  pad = E_PAD - E
  pad_i = jnp.arange(pad, dtype=jnp.int32)
  srcb = jnp.concatenate([src, pad_i % N])
  dstb = jnp.concatenate([dst, N + pad_i % (NACC - N)])
  z128 = jnp.zeros((NACC, D), jnp.float32)
  ones128 = jnp.ones((CHUNK, D), jnp.float32)

  x_pad = jnp.concatenate([x, jnp.zeros((N_PAD - N, D), jnp.float32)])
  src3 = srcb.reshape(NW, NCH, CHUNK)
  dst3 = dstb.reshape(NW, NCH, CHUNK)
  degp = _deg(dst3, z128, ones128)
  n1p = _agg(x, src3, dst3, z128)
  h1 = _tc_layer1(x_pad, n1p[0], n1p[1], degp[0], degp[1], W1s, W1n, b1s, b1n)
  n2p = _agg(h1, src3, dst3, z128)
  Wpc = jnp.concatenate([Wp[:D], Wp[D:]], axis=1)
  bvec = jnp.concatenate([jnp.zeros((1,), jnp.float32), bp]).reshape(1, 2)
  ab = _tc_layer2(h1, n2p[0], n2p[1], degp[0], degp[1], W2s, W2n, b2s, b2n,
                  Wpc, bvec)
  score = _edge_score(ab.reshape(N_PAD * 2), srcb, dstb)
  return score[:E].reshape(E, 1)
